# baseline (device time: 256964 ns/iter reference)
import functools

import jax
import jax.numpy as jnp
from jax import lax
from jax.experimental import pallas as pl
from jax.experimental.pallas import tpu as pltpu

B, S, H, Dh, Dr = 4, 256, 32, 128, 64
M = B * S
D = 4096
DC_SHARD = 128
SCALE = (Dh + Dr) ** -0.5


def _mm(x, w, out_dtype, bm, bn, bk):
    m, k = x.shape
    k2, n = w.shape
    assert k == k2 and m % bm == 0 and n % bn == 0 and k % bk == 0
    nk = k // bk

    def body(x_ref, w_ref, o_ref, acc_ref):
        @pl.when(pl.program_id(2) == 0)
        def _():
            acc_ref[...] = jnp.zeros_like(acc_ref)

        acc_ref[...] += jnp.dot(
            x_ref[...].astype(jnp.bfloat16),
            w_ref[...].astype(jnp.bfloat16),
            preferred_element_type=jnp.float32,
        )

        @pl.when(pl.program_id(2) == nk - 1)
        def _():
            o_ref[...] = acc_ref[...].astype(o_ref.dtype)

    return pl.pallas_call(
        body,
        grid=(m // bm, n // bn, nk),
        in_specs=[
            pl.BlockSpec((bm, bk), lambda i, j, kk: (i, kk)),
            pl.BlockSpec((bk, bn), lambda i, j, kk: (kk, j)),
        ],
        out_specs=pl.BlockSpec((bm, bn), lambda i, j, kk: (i, j)),
        out_shape=jax.ShapeDtypeStruct((m, n), out_dtype),
        scratch_shapes=[pltpu.VMEM((bm, bn), jnp.float32)],
    )(x, w)


def _exchange_kv(c_me, wuk, wuv):
    def body(c_ref, wuk_ref, wuv_ref, k_out, v_out,
             wuk_bf, wuv_bf, c_rx, wuk_rx, wuv_rx, send_sems, recv_sems):
        mx = lax.axis_index("x")
        my = lax.axis_index("y")
        mz = lax.axis_index("z")
        peer = (mx, 1 - my, mz)

        wuk_bf[...] = wuk_ref[...].astype(jnp.bfloat16)
        wuv_bf[...] = wuv_ref[...].astype(jnp.bfloat16)

        barrier_sem = pltpu.get_barrier_semaphore()
        pl.semaphore_signal(
            barrier_sem, inc=1, device_id=peer,
            device_id_type=pl.DeviceIdType.MESH,
        )
        pl.semaphore_wait(barrier_sem, 1)

        rdmas = []
        for i, (src, dst) in enumerate(
            [(c_ref, c_rx), (wuk_bf, wuk_rx), (wuv_bf, wuv_rx)]
        ):
            rdma = pltpu.make_async_remote_copy(
                src_ref=src,
                dst_ref=dst,
                send_sem=send_sems.at[i],
                recv_sem=recv_sems.at[i],
                device_id=peer,
                device_id_type=pl.DeviceIdType.MESH,
            )
            rdma.start()
            rdmas.append(rdma)

        c_local = c_ref[...]
        k_loc = jnp.dot(c_local, wuk_bf[...], preferred_element_type=jnp.float32)
        v_loc = jnp.dot(c_local, wuv_bf[...], preferred_element_type=jnp.float32)

        for rdma in rdmas:
            rdma.wait()

        c_p = c_rx[...]
        k_out[...] = (
            k_loc + jnp.dot(c_p, wuk_rx[...], preferred_element_type=jnp.float32)
        ).astype(jnp.bfloat16)
        v_out[...] = (
            v_loc + jnp.dot(c_p, wuv_rx[...], preferred_element_type=jnp.float32)
        ).astype(jnp.bfloat16)

    return pl.pallas_call(
        body,
        in_specs=[
            pl.BlockSpec(memory_space=pltpu.VMEM),
            pl.BlockSpec(memory_space=pltpu.VMEM),
            pl.BlockSpec(memory_space=pltpu.VMEM),
        ],
        out_specs=[
            pl.BlockSpec(memory_space=pltpu.VMEM),
            pl.BlockSpec(memory_space=pltpu.VMEM),
        ],
        out_shape=[
            jax.ShapeDtypeStruct((M, D), jnp.bfloat16),
            jax.ShapeDtypeStruct((M, D), jnp.bfloat16),
        ],
        scratch_shapes=[
            pltpu.VMEM((DC_SHARD, D), jnp.bfloat16),
            pltpu.VMEM((DC_SHARD, D), jnp.bfloat16),
            pltpu.VMEM((M, DC_SHARD), jnp.bfloat16),
            pltpu.VMEM((DC_SHARD, D), jnp.bfloat16),
            pltpu.VMEM((DC_SHARD, D), jnp.bfloat16),
            pltpu.SemaphoreType.DMA((3,)),
            pltpu.SemaphoreType.DMA((3,)),
        ],
        compiler_params=pltpu.CompilerParams(collective_id=0),
    )(c_me, wuk, wuv)


def _attention(q, k, v, qr, kr):
    def body(q_ref, k_ref, v_ref, qr_ref, kr_ref, o_ref):
        qv = q_ref[...]
        kv = k_ref[...]
        vv = v_ref[...]
        qrv = qr_ref[...]
        krv = kr_ref[...]
        nt_dims = (((1,), (1,)), ((), ()))
        for h in range(H):
            qh = qv[:, h * Dh:(h + 1) * Dh]
            kh = kv[:, h * Dh:(h + 1) * Dh]
            qrh = qrv[:, h * Dr:(h + 1) * Dr]
            s = (
                lax.dot_general(qh, kh, nt_dims,
                                preferred_element_type=jnp.float32)
                + lax.dot_general(qrh, krv, nt_dims,
                                  preferred_element_type=jnp.float32)
            ) * SCALE
            mmax = jnp.max(s, axis=1, keepdims=True)
            p = jnp.exp(s - mmax)
            p = p / jnp.sum(p, axis=1, keepdims=True)
            o = jnp.dot(p.astype(jnp.bfloat16), vv[:, h * Dh:(h + 1) * Dh],
                        preferred_element_type=jnp.float32)
            o_ref[:, h * Dh:(h + 1) * Dh] = o.astype(jnp.bfloat16)

    return pl.pallas_call(
        body,
        grid=(B,),
        in_specs=[
            pl.BlockSpec((S, D), lambda b: (b, 0)),
            pl.BlockSpec((S, D), lambda b: (b, 0)),
            pl.BlockSpec((S, D), lambda b: (b, 0)),
            pl.BlockSpec((S, H * Dr), lambda b: (b, 0)),
            pl.BlockSpec((S, Dr), lambda b: (b, 0)),
        ],
        out_specs=pl.BlockSpec((S, D), lambda b: (b, 0)),
        out_shape=jax.ShapeDtypeStruct((M, D), jnp.bfloat16),
    )(q, k, v, qr, kr)


def kernel(x, Wdkv, Wuk, Wuv, Wq, Wqr, Wkr, Wo):
    x_flat = x.reshape(M, D)

    c = _mm(x_flat, Wdkv, jnp.bfloat16, bm=1024, bn=128, bk=1024)
    q = _mm(x_flat, Wq, jnp.bfloat16, bm=1024, bn=1024, bk=1024)
    qr = _mm(x_flat, Wqr, jnp.bfloat16, bm=1024, bn=1024, bk=1024)
    kr = _mm(x_flat, Wkr, jnp.bfloat16, bm=1024, bn=64, bk=1024)

    k, v = _exchange_kv(c, Wuk, Wuv)

    o = _attention(q, k, v, qr, kr)

    out = _mm(o, Wo, jnp.float32, bm=1024, bn=1024, bk=1024)
    return out.reshape(B, S, D)


# device time: 232672 ns/iter; 1.1044x vs baseline; 1.1044x over previous
import functools

import jax
import jax.numpy as jnp
from jax import lax
from jax.experimental import pallas as pl
from jax.experimental.pallas import tpu as pltpu

B, S, H, Dh, Dr = 4, 256, 32, 128, 64
M = B * S
D = 4096
DC_SHARD = 128
SCALE = (Dh + Dr) ** -0.5


def _mm(x, w, out_dtype, bm, bn, bk, name="mm", out_scale=None):
    m, k = x.shape
    k2, n = w.shape
    assert k == k2 and m % bm == 0 and n % bn == 0 and k % bk == 0
    nk = k // bk

    def body(x_ref, w_ref, o_ref, acc_ref):
        @pl.when(pl.program_id(2) == 0)
        def _():
            acc_ref[...] = jnp.zeros_like(acc_ref)

        acc_ref[...] += jnp.dot(
            x_ref[...].astype(jnp.bfloat16),
            w_ref[...].astype(jnp.bfloat16),
            preferred_element_type=jnp.float32,
        )

        @pl.when(pl.program_id(2) == nk - 1)
        def _():
            acc = acc_ref[...]
            if out_scale is not None:
                acc = acc * out_scale
            o_ref[...] = acc.astype(o_ref.dtype)

    return pl.pallas_call(
        body,
        grid=(m // bm, n // bn, nk),
        in_specs=[
            pl.BlockSpec((bm, bk), lambda i, j, kk: (i, kk)),
            pl.BlockSpec((bk, bn), lambda i, j, kk: (kk, j)),
        ],
        out_specs=pl.BlockSpec((bm, bn), lambda i, j, kk: (i, j)),
        out_shape=jax.ShapeDtypeStruct((m, n), out_dtype),
        scratch_shapes=[pltpu.VMEM((bm, bn), jnp.float32)],
        name=name,
    )(x, w)


def _exchange_kv(c_me, wuk, wuv):
    def body(c_ref, wuk_ref, wuv_ref, k_out, v_out,
             wuk_bf, wuv_bf, c_rx, wuk_rx, wuv_rx, send_sems, recv_sems):
        mx = lax.axis_index("x")
        my = lax.axis_index("y")
        mz = lax.axis_index("z")
        peer = (mx, 1 - my, mz)

        wuk_bf[...] = wuk_ref[...].astype(jnp.bfloat16)
        wuv_bf[...] = wuv_ref[...].astype(jnp.bfloat16)

        barrier_sem = pltpu.get_barrier_semaphore()
        pl.semaphore_signal(
            barrier_sem, inc=1, device_id=peer,
            device_id_type=pl.DeviceIdType.MESH,
        )
        pl.semaphore_wait(barrier_sem, 1)

        rdmas = []
        for i, (src, dst) in enumerate(
            [(c_ref, c_rx), (wuk_bf, wuk_rx), (wuv_bf, wuv_rx)]
        ):
            rdma = pltpu.make_async_remote_copy(
                src_ref=src,
                dst_ref=dst,
                send_sem=send_sems.at[i],
                recv_sem=recv_sems.at[i],
                device_id=peer,
                device_id_type=pl.DeviceIdType.MESH,
            )
            rdma.start()
            rdmas.append(rdma)

        c_local = c_ref[...]
        k_loc = jnp.dot(c_local, wuk_bf[...], preferred_element_type=jnp.float32)
        v_loc = jnp.dot(c_local, wuv_bf[...], preferred_element_type=jnp.float32)

        for rdma in rdmas:
            rdma.wait()

        c_p = c_rx[...]
        k_out[...] = (
            k_loc + jnp.dot(c_p, wuk_rx[...], preferred_element_type=jnp.float32)
        ).astype(jnp.bfloat16)
        v_out[...] = (
            v_loc + jnp.dot(c_p, wuv_rx[...], preferred_element_type=jnp.float32)
        ).astype(jnp.bfloat16)

    return pl.pallas_call(
        body,
        in_specs=[
            pl.BlockSpec(memory_space=pltpu.VMEM),
            pl.BlockSpec(memory_space=pltpu.VMEM),
            pl.BlockSpec(memory_space=pltpu.VMEM),
        ],
        out_specs=[
            pl.BlockSpec(memory_space=pltpu.VMEM),
            pl.BlockSpec(memory_space=pltpu.VMEM),
        ],
        out_shape=[
            jax.ShapeDtypeStruct((M, D), jnp.bfloat16),
            jax.ShapeDtypeStruct((M, D), jnp.bfloat16),
        ],
        scratch_shapes=[
            pltpu.VMEM((DC_SHARD, D), jnp.bfloat16),
            pltpu.VMEM((DC_SHARD, D), jnp.bfloat16),
            pltpu.VMEM((M, DC_SHARD), jnp.bfloat16),
            pltpu.VMEM((DC_SHARD, D), jnp.bfloat16),
            pltpu.VMEM((DC_SHARD, D), jnp.bfloat16),
            pltpu.SemaphoreType.DMA((3,)),
            pltpu.SemaphoreType.DMA((3,)),
        ],
        compiler_params=pltpu.CompilerParams(collective_id=0),
        name="exchange_kv",
    )(c_me, wuk, wuv)


def _attention(q, k, v, qr, kr):
    def body(q_ref, k_ref, v_ref, qr_ref, kr_ref, o_ref):
        qv = q_ref[...]
        kv = k_ref[...]
        vv = v_ref[...]
        qrv = qr_ref[...]
        krv = kr_ref[...]
        nt_dims = (((1,), (1,)), ((), ()))
        for h in range(H):
            qh = qv[:, h * Dh:(h + 1) * Dh]
            kh = kv[:, h * Dh:(h + 1) * Dh]
            qrh = qrv[:, h * Dr:(h + 1) * Dr]
            s = lax.dot_general(qh, kh, nt_dims,
                                preferred_element_type=jnp.float32)
            s += lax.dot_general(qrh, krv, nt_dims,
                                 preferred_element_type=jnp.float32)
            p = jnp.exp(s)
            denom = jnp.sum(p, axis=1, keepdims=True)
            o = jnp.dot(p.astype(jnp.bfloat16), vv[:, h * Dh:(h + 1) * Dh],
                        preferred_element_type=jnp.float32)
            o_ref[:, h * Dh:(h + 1) * Dh] = (o / denom).astype(jnp.bfloat16)

    return pl.pallas_call(
        body,
        grid=(B,),
        in_specs=[
            pl.BlockSpec((S, D), lambda b: (b, 0)),
            pl.BlockSpec((S, D), lambda b: (b, 0)),
            pl.BlockSpec((S, D), lambda b: (b, 0)),
            pl.BlockSpec((S, H * Dr), lambda b: (b, 0)),
            pl.BlockSpec((S, Dr), lambda b: (b, 0)),
        ],
        out_specs=pl.BlockSpec((S, D), lambda b: (b, 0)),
        out_shape=jax.ShapeDtypeStruct((M, D), jnp.bfloat16),
        name="attention",
    )(q, k, v, qr, kr)


def kernel(x, Wdkv, Wuk, Wuv, Wq, Wqr, Wkr, Wo):
    x_flat = x.reshape(M, D)

    c = _mm(x_flat, Wdkv, jnp.bfloat16, bm=1024, bn=128, bk=1024, name="mm_c")
    q = _mm(x_flat, Wq, jnp.bfloat16, bm=1024, bn=1024, bk=1024, name="mm_q",
            out_scale=SCALE)
    qr = _mm(x_flat, Wqr, jnp.bfloat16, bm=1024, bn=1024, bk=1024,
             name="mm_qr", out_scale=SCALE)
    kr = _mm(x_flat, Wkr, jnp.bfloat16, bm=1024, bn=64, bk=1024, name="mm_kr")

    k, v = _exchange_kv(c, Wuk, Wuv)

    o = _attention(q, k, v, qr, kr)

    out = _mm(o, Wo, jnp.float32, bm=1024, bn=1024, bk=1024, name="mm_out")
    return out.reshape(B, S, D)
